# Initial kernel scaffold; baseline (speedup 1.0000x reference)
import jax
import jax.numpy as jnp
from jax.experimental import pallas as pl


def kernel(Za, Dij, idx_i, idx_j, c6ab, rcov, r2r4):
    raise NotImplementedError("write your pallas kernel here")


# speedup vs baseline: 49.0843x; 49.0843x over previous
import functools

import jax
import jax.numpy as jnp
from jax import lax
from jax.experimental import pallas as pl
from jax.experimental.pallas import tpu as pltpu
from jax.experimental.pallas import tpu_sc as plsc

D3_MAXC = 5
NZ = 95
NPAIR = NZ * NZ
K1 = 16.0
K3 = -4.0
S6 = 1.0
S8 = 0.9171
A1 = 0.3385
A2 = 2.883
INV_BOHR = 1.0 / 0.5291772108
N_NODES = 100000
N_EDGES = 1600000

NC = 2
NS = 16
NW = NC * NS
L = 16

NPAD = 100352
SLICE = NPAD // NS
CSL = NPAD // NW
EPW = N_EDGES // NW

C1 = 400
NCH1 = EPW // C1
C2 = 80
NCH2 = EPW // C2
ROWW = 32

_mesh = plsc.VectorSubcoreMesh(core_axis_name="c", subcore_axis_name="s")


def _cvec(n):
    return jnp.full((L,), n, jnp.int32)


def _sqrt(x):
    i = lax.bitcast_convert_type(x, jnp.int32)
    i = 0x5F3759DF - lax.shift_right_logical(i, 1)
    y = lax.bitcast_convert_type(i, jnp.float32)
    for _ in range(3):
        y = y * (1.5 - 0.5 * x * y * y)
    return x * y


def _zero_fill(buf, nwords):
    z = jnp.zeros((L,), jnp.float32)

    def body(i, x):
        buf[pl.ds(i * L, L)] = z
        return x

    lax.fori_loop(0, nwords // L, body, 0)


@functools.partial(
    pl.kernel,
    out_type=(
        jax.ShapeDtypeStruct((NC * NPAD,), jnp.float32),
        jax.ShapeDtypeStruct((N_EDGES,), jnp.int32),
        jax.ShapeDtypeStruct((N_EDGES,), jnp.float32),
    ),
    mesh=_mesh,
    compiler_params=pltpu.CompilerParams(needs_layout_passes=False),
    scratch_types=dict(
        za_v=pltpu.VMEM((N_NODES,), jnp.int32),
        rcv=pltpu.VMEM((96,), jnp.float32),
        r4v=pltpu.VMEM((96,), jnp.float32),
        ii_b=pltpu.VMEM((2 * C1,), jnp.int32),
        jj_b=pltpu.VMEM((2 * C1,), jnp.int32),
        d_b=pltpu.VMEM((2 * C1,), jnp.float32),
        pid_b=pltpu.VMEM((2 * C1,), jnp.int32),
        rr4_b=pltpu.VMEM((2 * C1,), jnp.float32),
        damp_b=pltpu.VMEM((2 * C1,), jnp.float32),
        sii=pltpu.VMEM((2 * C1,), jnp.int32),
        zb=pltpu.VMEM((SLICE,), jnp.float32),
        nc_sh=pltpu.VMEM_SHARED((NPAD,), jnp.float32),
        lsem=pltpu.SemaphoreType.DMA,
        osem=pltpu.SemaphoreType.DMA,
        ssem=pltpu.SemaphoreType.DMA,
    ),
)
def _phase_a(za, dij, idx_i, idx_j, rcov, r2r4, nc_out, pid_out, rr4_out,
             za_v, rcv, r4v, ii_b, jj_b, d_b, pid_b, rr4_b, damp_b, sii, zb,
             nc_sh, lsem, osem, ssem):
    c = lax.axis_index("c")
    s = lax.axis_index("s")
    base_e = (c * NS + s) * EPW

    _zero_fill(zb, SLICE)
    pltpu.sync_copy(zb, nc_sh.at[pl.ds(s * SLICE, SLICE)])
    pltpu.sync_copy(za, za_v)
    pltpu.sync_copy(rcov, rcv)
    pltpu.sync_copy(r2r4, r4v)
    plsc.subcore_barrier()

    def lin_dma(g, slot):
        src = pl.ds(base_e + g * C1, C1)
        dst = pl.ds(slot * C1, C1)
        return (
            pltpu.make_async_copy(idx_i.at[src], ii_b.at[dst], lsem),
            pltpu.make_async_copy(idx_j.at[src], jj_b.at[dst], lsem),
            pltpu.make_async_copy(dij.at[src], d_b.at[dst], lsem),
        )

    for d in lin_dma(0, 0):
        d.start()

    def chunk(g, x):
        p = jnp.bitwise_and(g, 1)
        for d in lin_dma(g, p):
            d.wait()

        @pl.when(g + 1 < NCH1)
        def _():
            for d in lin_dma(g + 1, 1 - p):
                d.start()

        for g5 in range(C1 // L // 5):
            offs = [p * C1 + (g5 * 5 + t) * L for t in range(5)]
            iis = [ii_b[pl.ds(o, L)] for o in offs]
            jjs = [jj_b[pl.ds(o, L)] for o in offs]
            dvs = [d_b[pl.ds(o, L)] for o in offs]
            zis = [plsc.load_gather(za_v, [x]) for x in iis]
            zjs = [plsc.load_gather(za_v, [x]) for x in jjs]
            rcis = [plsc.load_gather(rcv, [z]) for z in zis]
            rcjs = [plsc.load_gather(rcv, [z]) for z in zjs]
            r4is = [plsc.load_gather(r4v, [z]) for z in zis]
            r4js = [plsc.load_gather(r4v, [z]) for z in zjs]
            for t in range(5):
                o = offs[t]
                rr = (rcis[t] + rcjs[t]) / (dvs[t] * INV_BOHR)
                damp = 1.0 / (1.0 + jnp.exp(K1 - K1 * rr))
                pid_b[pl.ds(o, L)] = zis[t] * NZ + zjs[t]
                rr4_b[pl.ds(o, L)] = r4is[t] * r4js[t]
                damp_b[pl.ds(o, L)] = damp
                sii[pl.ds(o, L)] = iis[t]

        def out_dma(gg, pp):
            dst = pl.ds(base_e + gg * C1, C1)
            return (
                pltpu.make_async_copy(
                    pid_b.at[pl.ds(pp * C1, C1)], pid_out.at[dst], osem),
                pltpu.make_async_copy(
                    rr4_b.at[pl.ds(pp * C1, C1)], rr4_out.at[dst], osem),
            )

        def sc_dma(pp):
            return pltpu.make_async_copy(
                damp_b.at[pl.ds(pp * C1, C1)],
                nc_sh.at[sii.at[pl.ds(pp * C1, C1)]], ssem)

        @pl.when(g > 0)
        def _():
            for d in out_dma(g - 1, 1 - p):
                d.wait()
            sc_dma(1 - p).wait()

        for d in out_dma(g, p):
            d.start()
        sc_dma(p).start(add=True)
        return x

    lax.fori_loop(0, NCH1, chunk, 0)

    pf = jnp.bitwise_and(NCH1 - 1, 1)
    dst = pl.ds(base_e + (NCH1 - 1) * C1, C1)
    pltpu.make_async_copy(
        pid_b.at[pl.ds(pf * C1, C1)], pid_out.at[dst], osem).wait()
    pltpu.make_async_copy(
        rr4_b.at[pl.ds(pf * C1, C1)], rr4_out.at[dst], osem).wait()
    pltpu.make_async_copy(
        damp_b.at[pl.ds(pf * C1, C1)],
        nc_sh.at[sii.at[pl.ds(pf * C1, C1)]], ssem).wait()
    plsc.subcore_barrier()
    sl = pl.ds(s * SLICE, SLICE)
    pltpu.sync_copy(nc_sh.at[sl], zb)
    pltpu.sync_copy(zb, nc_out.at[pl.ds(c * NPAD + s * SLICE, SLICE)])


@functools.partial(
    pl.kernel,
    out_type=jax.ShapeDtypeStruct((NC * NPAD,), jnp.float32),
    mesh=_mesh,
    compiler_params=pltpu.CompilerParams(
        needs_layout_passes=False, use_tc_tiling_on_sc=False),
    scratch_types=dict(
        r_c6=pltpu.VMEM((2 * C2, ROWW), jnp.float32),
        r_x=pltpu.VMEM((2 * C2, ROWW), jnp.float32),
        r_y=pltpu.VMEM((2 * C2, ROWW), jnp.float32),
        r_b=pltpu.VMEM((2 * C2, ROWW), jnp.float32),
        nci_b=pltpu.VMEM((2 * C2,), jnp.float32),
        ncj_b=pltpu.VMEM((2 * C2,), jnp.float32),
        ii_b=pltpu.VMEM((3 * C2,), jnp.int32),
        jj_b=pltpu.VMEM((3 * C2,), jnp.int32),
        d_b=pltpu.VMEM((3 * C2,), jnp.float32),
        rr4_b=pltpu.VMEM((3 * C2,), jnp.float32),
        pid_b=pltpu.VMEM((3 * C2,), jnp.int32),
        e_b=pltpu.VMEM((2 * C2,), jnp.float32),
        sii=pltpu.VMEM((2 * C2,), jnp.int32),
        zb=pltpu.VMEM((SLICE,), jnp.float32),
        e_sh=pltpu.VMEM_SHARED((NPAD,), jnp.float32),
        lsem=pltpu.SemaphoreType.DMA,
        gsem=pltpu.SemaphoreType.DMA,
        ssem=pltpu.SemaphoreType.DMA,
    ),
)
def _phase_b(dij, idx_i, idx_j, pid, rr4, t_c6, t_x, t_y, t_b, nc_full, e_out,
             r_c6, r_x, r_y, r_b, nci_b, ncj_b, ii_b, jj_b, d_b, rr4_b,
             pid_b, e_b, sii, zb, e_sh, lsem, gsem, ssem):
    c = lax.axis_index("c")
    s = lax.axis_index("s")
    base_e = (c * NS + s) * EPW

    _zero_fill(zb, SLICE)
    pltpu.sync_copy(zb, e_sh.at[pl.ds(s * SLICE, SLICE)])
    plsc.subcore_barrier()

    def lin_dma(g, slot):
        src = pl.ds(base_e + g * C2, C2)
        dst = pl.ds(slot * C2, C2)
        return (
            pltpu.make_async_copy(idx_i.at[src], ii_b.at[dst], lsem),
            pltpu.make_async_copy(idx_j.at[src], jj_b.at[dst], lsem),
            pltpu.make_async_copy(dij.at[src], d_b.at[dst], lsem),
            pltpu.make_async_copy(rr4.at[src], rr4_b.at[dst], lsem),
            pltpu.make_async_copy(pid.at[src], pid_b.at[dst], lsem),
        )

    lane = lax.iota(jnp.int32, L)

    def gat_dma(slot, p):
        isl = pid_b.at[pl.ds(slot * C2, C2)]
        dsl = pl.ds(p * C2, C2)
        return (
            pltpu.make_async_copy(t_c6.at[isl], r_c6.at[dsl], gsem),
            pltpu.make_async_copy(t_x.at[isl], r_x.at[dsl], gsem),
            pltpu.make_async_copy(t_y.at[isl], r_y.at[dsl], gsem),
            pltpu.make_async_copy(t_b.at[isl], r_b.at[dsl], gsem),
            pltpu.make_async_copy(
                nc_full.at[ii_b.at[pl.ds(slot * C2, C2)]],
                nci_b.at[dsl], gsem),
            pltpu.make_async_copy(
                nc_full.at[jj_b.at[pl.ds(slot * C2, C2)]],
                ncj_b.at[dsl], gsem),
        )

    for d in lin_dma(0, 0):
        d.start()
    for d in lin_dma(0, 0):
        d.wait()
    for d in gat_dma(0, 0):
        d.start()
    for d in lin_dma(1, 1):
        d.start()

    def chunk(g, x):
        p = jnp.bitwise_and(g, 1)
        q = lax.rem(g, 3)
        for d in gat_dma(q, p):
            d.wait()

        @pl.when(g + 1 < NCH2)
        def _():
            q1 = lax.rem(g + 1, 3)
            for d in lin_dma(g + 1, q1):
                d.wait()
            for d in gat_dma(q1, 1 - p):
                d.start()

        @pl.when(g + 2 < NCH2)
        def _():
            q2 = lax.rem(g + 2, 3)
            for d in lin_dma(g + 2, q2):
                d.start()

        for v in range(C2 // L):
            off = q * C2 + v * L
            ii_v = ii_b[pl.ds(off, L)]
            d_v = d_b[pl.ds(off, L)]
            rr4_v = rr4_b[pl.ds(off, L)]
            nci = nci_b[pl.ds(p * C2 + v * L, L)]
            ncj = ncj_b[pl.ds(p * C2 + v * L, L)]
            row = lane + (p * C2 + v * L)

            tks = []
            cols = []
            tmaxs = [None] * 4
            for k in range(25):
                col = _cvec(k)
                cols.append(col)
                xk = plsc.load_gather(r_x, [row, col])
                yk = plsc.load_gather(r_y, [row, col])
                bk = plsc.load_gather(r_b, [row, col])
                tk = (bk + xk * nci) + yk * ncj
                tks.append(tk)
                a = k & 3
                tmaxs[a] = tk if tmaxs[a] is None else jnp.maximum(tmaxs[a], tk)
            sh = jnp.maximum(jnp.maximum(tmaxs[0], tmaxs[1]),
                             jnp.maximum(tmaxs[2], tmaxs[3]))
            ses = [None] * 4
            scs = [None] * 4
            for k in range(25):
                e = jnp.exp(tks[k] - sh)
                c6k = plsc.load_gather(r_c6, [row, cols[k]])
                ec = e * c6k
                a = k & 3
                ses[a] = e if ses[a] is None else ses[a] + e
                scs[a] = ec if scs[a] is None else scs[a] + ec
            se = (ses[0] + ses[1]) + (ses[2] + ses[3])
            sc6 = (scs[0] + scs[1]) + (scs[2] + scs[3])
            c6 = sc6 / se
            c8 = 3.0 * c6 * rr4_v
            db = d_v * INV_BOHR
            r2 = db * db
            r6 = r2 * r2 * r2
            r8 = r6 * r2
            tmp = A1 * _sqrt(c8 / (c6 + 1e-10) + 1e-10) + A2
            t2 = tmp * tmp
            t6 = t2 * t2 * t2
            t8 = t6 * t2
            ev = (-0.5 * S6) * c6 / (r6 + t6) + (-0.5 * S8) * c8 / (r8 + t8)
            e_b[pl.ds(p * C2 + v * L, L)] = ev
            sii[pl.ds(p * C2 + v * L, L)] = ii_v

        def sc_dma(pp):
            return pltpu.make_async_copy(
                e_b.at[pl.ds(pp * C2, C2)],
                e_sh.at[sii.at[pl.ds(pp * C2, C2)]], ssem)

        @pl.when(g > 0)
        def _():
            sc_dma(1 - p).wait()

        sc_dma(p).start(add=True)
        return x

    lax.fori_loop(0, NCH2, chunk, 0)

    pf = jnp.bitwise_and(NCH2 - 1, 1)
    pltpu.make_async_copy(
        e_b.at[pl.ds(pf * C2, C2)],
        e_sh.at[sii.at[pl.ds(pf * C2, C2)]], ssem).wait()
    plsc.subcore_barrier()
    sl = pl.ds(s * SLICE, SLICE)
    pltpu.sync_copy(e_sh.at[sl], zb)
    pltpu.sync_copy(zb, e_out.at[pl.ds(c * NPAD + s * SLICE, SLICE)])


@functools.partial(
    pl.kernel,
    out_type=jax.ShapeDtypeStruct((NPAD,), jnp.float32),
    mesh=_mesh,
    compiler_params=pltpu.CompilerParams(needs_layout_passes=False),
    scratch_types=dict(
        b0=pltpu.VMEM((CSL,), jnp.float32),
        b1=pltpu.VMEM((CSL,), jnp.float32),
    ),
)
def _phase_c(e_part, e_out, b0, b1):
    w = lax.axis_index("c") * NS + lax.axis_index("s")
    pltpu.sync_copy(e_part.at[pl.ds(w * CSL, CSL)], b0)
    pltpu.sync_copy(e_part.at[pl.ds(NPAD + w * CSL, CSL)], b1)

    def add16(i, x):
        s16 = pl.ds(i * L, L)
        b0[s16] = b0[s16] + b1[s16]
        return x

    lax.fori_loop(0, CSL // L, add16, 0)
    pltpu.sync_copy(b0, e_out.at[pl.ds(w * CSL, CSL)])


def kernel(Za, Dij, idx_i, idx_j, c6ab, rcov, r2r4):
    comp = c6ab.reshape(NPAIR, D3_MAXC * D3_MAXC, 3)
    c6t = comp[:, :, 0]
    cni_t = comp[:, :, 1]
    cnj_t = comp[:, :, 2]
    pad7 = ((0, 0), (0, ROWW - 25))
    t_c6 = jnp.pad(c6t, pad7)
    t_x = jnp.pad((-2.0 * K3) * cni_t, pad7)
    t_y = jnp.pad((-2.0 * K3) * cnj_t, pad7)
    t_b = jnp.pad(K3 * (cni_t * cni_t + cnj_t * cnj_t), pad7)
    rcov_p = jnp.pad(rcov, (0, 1))
    r2r4_p = jnp.pad(r2r4, (0, 1))
    nc_part, pid, rr4 = _phase_a(Za, Dij, idx_i, idx_j, rcov_p, r2r4_p)
    nc_full = _phase_c(nc_part)
    e_part = _phase_b(Dij, idx_i, idx_j, pid, rr4,
                      t_c6, t_x, t_y, t_b, nc_full)
    e_full = _phase_c(e_part)
    return e_full[:N_NODES]


# speedup vs baseline: 51.6359x; 1.0520x over previous
import functools

import jax
import jax.numpy as jnp
from jax import lax
from jax.experimental import pallas as pl
from jax.experimental.pallas import tpu as pltpu
from jax.experimental.pallas import tpu_sc as plsc

D3_MAXC = 5
NZ = 95
NPAIR = NZ * NZ
K1 = 16.0
K3 = -4.0
S6 = 1.0
S8 = 0.9171
A1 = 0.3385
A2 = 2.883
INV_BOHR = 1.0 / 0.5291772108
N_NODES = 100000
N_EDGES = 1600000

NC = 2
NS = 16
NW = NC * NS
L = 16

NPAD = 100352
SLICE = NPAD // NS
CSL = NPAD // NW
EPW = N_EDGES // NW

C1 = 400
NCH1 = EPW // C1
C2 = 80
NCH2 = EPW // C2
ROWW = 32

_mesh = plsc.VectorSubcoreMesh(core_axis_name="c", subcore_axis_name="s")


def _cvec(n):
    return jnp.full((L,), n, jnp.int32)


def _zero_fill(buf, nwords):
    z = jnp.zeros((L,), jnp.float32)

    def body(i, x):
        buf[pl.ds(i * L, L)] = z
        return x

    lax.fori_loop(0, nwords // L, body, 0)


@functools.partial(
    pl.kernel,
    out_type=(
        jax.ShapeDtypeStruct((NC * NPAD,), jnp.float32),
        jax.ShapeDtypeStruct((N_EDGES,), jnp.int32),
        jax.ShapeDtypeStruct((N_EDGES,), jnp.float32),
    ),
    mesh=_mesh,
    compiler_params=pltpu.CompilerParams(needs_layout_passes=False),
    scratch_types=dict(
        za_v=pltpu.VMEM((N_NODES,), jnp.int32),
        rcv=pltpu.VMEM((96,), jnp.float32),
        r4v=pltpu.VMEM((96,), jnp.float32),
        ii_b=pltpu.VMEM((2 * C1,), jnp.int32),
        jj_b=pltpu.VMEM((2 * C1,), jnp.int32),
        d_b=pltpu.VMEM((2 * C1,), jnp.float32),
        pid_b=pltpu.VMEM((2 * C1,), jnp.int32),
        rr4_b=pltpu.VMEM((2 * C1,), jnp.float32),
        damp_b=pltpu.VMEM((2 * C1,), jnp.float32),
        sii=pltpu.VMEM((2 * C1,), jnp.int32),
        zb=pltpu.VMEM((SLICE,), jnp.float32),
        nc_sh=pltpu.VMEM_SHARED((NPAD,), jnp.float32),
        lsem=pltpu.SemaphoreType.DMA,
        osem=pltpu.SemaphoreType.DMA,
        ssem=pltpu.SemaphoreType.DMA,
    ),
)
def _phase_a(za, dij, idx_i, idx_j, rcov, r2r4, nc_out, pid_out, rr4_out,
             za_v, rcv, r4v, ii_b, jj_b, d_b, pid_b, rr4_b, damp_b, sii, zb,
             nc_sh, lsem, osem, ssem):
    c = lax.axis_index("c")
    s = lax.axis_index("s")
    base_e = (c * NS + s) * EPW

    _zero_fill(zb, SLICE)
    pltpu.sync_copy(zb, nc_sh.at[pl.ds(s * SLICE, SLICE)])
    pltpu.sync_copy(za, za_v)
    pltpu.sync_copy(rcov, rcv)
    pltpu.sync_copy(r2r4, r4v)
    plsc.subcore_barrier()

    def lin_dma(g, slot):
        src = pl.ds(base_e + g * C1, C1)
        dst = pl.ds(slot * C1, C1)
        return (
            pltpu.make_async_copy(idx_i.at[src], ii_b.at[dst], lsem),
            pltpu.make_async_copy(idx_j.at[src], jj_b.at[dst], lsem),
            pltpu.make_async_copy(dij.at[src], d_b.at[dst], lsem),
        )

    for d in lin_dma(0, 0):
        d.start()

    def chunk(g, x):
        p = jnp.bitwise_and(g, 1)
        for d in lin_dma(g, p):
            d.wait()

        @pl.when(g + 1 < NCH1)
        def _():
            for d in lin_dma(g + 1, 1 - p):
                d.start()

        for g5 in range(C1 // L // 5):
            offs = [p * C1 + (g5 * 5 + t) * L for t in range(5)]
            iis = [ii_b[pl.ds(o, L)] for o in offs]
            jjs = [jj_b[pl.ds(o, L)] for o in offs]
            dvs = [d_b[pl.ds(o, L)] for o in offs]
            zis = [plsc.load_gather(za_v, [x]) for x in iis]
            zjs = [plsc.load_gather(za_v, [x]) for x in jjs]
            rcis = [plsc.load_gather(rcv, [z]) for z in zis]
            rcjs = [plsc.load_gather(rcv, [z]) for z in zjs]
            r4is = [plsc.load_gather(r4v, [z]) for z in zis]
            r4js = [plsc.load_gather(r4v, [z]) for z in zjs]
            for t in range(5):
                o = offs[t]
                rr = (rcis[t] + rcjs[t]) / (dvs[t] * INV_BOHR)
                damp = 1.0 / (1.0 + jnp.exp(K1 - K1 * rr))
                pid_b[pl.ds(o, L)] = zis[t] * NZ + zjs[t]
                rr4_b[pl.ds(o, L)] = r4is[t] * r4js[t]
                damp_b[pl.ds(o, L)] = damp
                sii[pl.ds(o, L)] = iis[t]

        def out_dma(gg, pp):
            dst = pl.ds(base_e + gg * C1, C1)
            return (
                pltpu.make_async_copy(
                    pid_b.at[pl.ds(pp * C1, C1)], pid_out.at[dst], osem),
                pltpu.make_async_copy(
                    rr4_b.at[pl.ds(pp * C1, C1)], rr4_out.at[dst], osem),
            )

        def sc_dma(pp):
            return pltpu.make_async_copy(
                damp_b.at[pl.ds(pp * C1, C1)],
                nc_sh.at[sii.at[pl.ds(pp * C1, C1)]], ssem)

        @pl.when(g > 0)
        def _():
            for d in out_dma(g - 1, 1 - p):
                d.wait()
            sc_dma(1 - p).wait()

        for d in out_dma(g, p):
            d.start()
        sc_dma(p).start(add=True)
        return x

    lax.fori_loop(0, NCH1, chunk, 0)

    pf = jnp.bitwise_and(NCH1 - 1, 1)
    dst = pl.ds(base_e + (NCH1 - 1) * C1, C1)
    pltpu.make_async_copy(
        pid_b.at[pl.ds(pf * C1, C1)], pid_out.at[dst], osem).wait()
    pltpu.make_async_copy(
        rr4_b.at[pl.ds(pf * C1, C1)], rr4_out.at[dst], osem).wait()
    pltpu.make_async_copy(
        damp_b.at[pl.ds(pf * C1, C1)],
        nc_sh.at[sii.at[pl.ds(pf * C1, C1)]], ssem).wait()
    plsc.subcore_barrier()
    sl = pl.ds(s * SLICE, SLICE)
    pltpu.sync_copy(nc_sh.at[sl], zb)
    pltpu.sync_copy(zb, nc_out.at[pl.ds(c * NPAD + s * SLICE, SLICE)])


@functools.partial(
    pl.kernel,
    out_type=jax.ShapeDtypeStruct((NC * NPAD,), jnp.float32),
    mesh=_mesh,
    compiler_params=pltpu.CompilerParams(
        needs_layout_passes=False, use_tc_tiling_on_sc=False),
    scratch_types=dict(
        r_c6=pltpu.VMEM((3 * C2, ROWW), jnp.float32),
        r_x=pltpu.VMEM((3 * C2, ROWW), jnp.float32),
        r_y=pltpu.VMEM((3 * C2, ROWW), jnp.float32),
        r_b=pltpu.VMEM((3 * C2, ROWW), jnp.float32),
        nci_b=pltpu.VMEM((3 * C2,), jnp.float32),
        ncj_b=pltpu.VMEM((3 * C2,), jnp.float32),
        ii_b=pltpu.VMEM((5 * C2,), jnp.int32),
        jj_b=pltpu.VMEM((5 * C2,), jnp.int32),
        d_b=pltpu.VMEM((5 * C2,), jnp.float32),
        rr4_b=pltpu.VMEM((5 * C2,), jnp.float32),
        pid_b=pltpu.VMEM((5 * C2,), jnp.int32),
        e_b=pltpu.VMEM((2 * C2,), jnp.float32),
        sii=pltpu.VMEM((2 * C2,), jnp.int32),
        zb=pltpu.VMEM((SLICE,), jnp.float32),
        e_sh=pltpu.VMEM_SHARED((NPAD,), jnp.float32),
        lsem=pltpu.SemaphoreType.DMA,
        gsem=pltpu.SemaphoreType.DMA,
        ssem=pltpu.SemaphoreType.DMA,
    ),
)
def _phase_b(dij, idx_i, idx_j, pid, rr4, t_c6, t_x, t_y, t_b, nc_full, e_out,
             r_c6, r_x, r_y, r_b, nci_b, ncj_b, ii_b, jj_b, d_b, rr4_b,
             pid_b, e_b, sii, zb, e_sh, lsem, gsem, ssem):
    c = lax.axis_index("c")
    s = lax.axis_index("s")
    base_e = (c * NS + s) * EPW

    _zero_fill(zb, SLICE)
    pltpu.sync_copy(zb, e_sh.at[pl.ds(s * SLICE, SLICE)])
    plsc.subcore_barrier()

    def lin_dma(g, slot):
        src = pl.ds(base_e + g * C2, C2)
        dst = pl.ds(slot * C2, C2)
        return (
            pltpu.make_async_copy(idx_i.at[src], ii_b.at[dst], lsem),
            pltpu.make_async_copy(idx_j.at[src], jj_b.at[dst], lsem),
            pltpu.make_async_copy(dij.at[src], d_b.at[dst], lsem),
            pltpu.make_async_copy(rr4.at[src], rr4_b.at[dst], lsem),
            pltpu.make_async_copy(pid.at[src], pid_b.at[dst], lsem),
        )

    lane = lax.iota(jnp.int32, L)

    def gat_dma(slot, p):
        isl = pid_b.at[pl.ds(slot * C2, C2)]
        dsl = pl.ds(p * C2, C2)
        return (
            pltpu.make_async_copy(t_c6.at[isl], r_c6.at[dsl], gsem),
            pltpu.make_async_copy(t_x.at[isl], r_x.at[dsl], gsem),
            pltpu.make_async_copy(t_y.at[isl], r_y.at[dsl], gsem),
            pltpu.make_async_copy(t_b.at[isl], r_b.at[dsl], gsem),
            pltpu.make_async_copy(
                nc_full.at[ii_b.at[pl.ds(slot * C2, C2)]],
                nci_b.at[dsl], gsem),
            pltpu.make_async_copy(
                nc_full.at[jj_b.at[pl.ds(slot * C2, C2)]],
                ncj_b.at[dsl], gsem),
        )

    for t in range(4):
        for d in lin_dma(t, t):
            d.start()
    for d in lin_dma(0, 0):
        d.wait()
    for d in gat_dma(0, 0):
        d.start()
    for d in lin_dma(1, 1):
        d.wait()
    for d in gat_dma(1, 1):
        d.start()

    def chunk(g, x):
        p = jnp.bitwise_and(g, 1)
        ql = lax.rem(g, 5)
        qg = lax.rem(g, 3)
        for d in gat_dma(ql, qg):
            d.wait()

        @pl.when(g + 2 < NCH2)
        def _():
            ql2 = lax.rem(g + 2, 5)
            qg2 = lax.rem(g + 2, 3)
            for d in lin_dma(g + 2, ql2):
                d.wait()
            for d in gat_dma(ql2, qg2):
                d.start()

        @pl.when(g + 4 < NCH2)
        def _():
            for d in lin_dma(g + 4, lax.rem(g + 4, 5)):
                d.start()

        for v in range(C2 // L):
            off = ql * C2 + v * L
            ii_v = ii_b[pl.ds(off, L)]
            d_v = d_b[pl.ds(off, L)]
            rr4_v = rr4_b[pl.ds(off, L)]
            nci = nci_b[pl.ds(qg * C2 + v * L, L)]
            ncj = ncj_b[pl.ds(qg * C2 + v * L, L)]
            row = lane + (qg * C2 + v * L)

            tks = []
            cols = []
            tmaxs = [None] * 4
            for k in range(25):
                col = _cvec(k)
                cols.append(col)
                xk = plsc.load_gather(r_x, [row, col])
                yk = plsc.load_gather(r_y, [row, col])
                bk = plsc.load_gather(r_b, [row, col])
                tk = (bk + xk * nci) + yk * ncj
                tks.append(tk)
                a = k & 3
                tmaxs[a] = tk if tmaxs[a] is None else jnp.maximum(tmaxs[a], tk)
            sh = jnp.maximum(jnp.maximum(tmaxs[0], tmaxs[1]),
                             jnp.maximum(tmaxs[2], tmaxs[3]))
            ses = [None] * 4
            scs = [None] * 4
            for k in range(25):
                e = jnp.exp(tks[k] - sh)
                c6k = plsc.load_gather(r_c6, [row, cols[k]])
                ec = e * c6k
                a = k & 3
                ses[a] = e if ses[a] is None else ses[a] + e
                scs[a] = ec if scs[a] is None else scs[a] + ec
            se = (ses[0] + ses[1]) + (ses[2] + ses[3])
            sc6 = (scs[0] + scs[1]) + (scs[2] + scs[3])
            c6 = sc6 / se
            c8 = 3.0 * c6 * (rr4_v * rr4_v)
            db = d_v * INV_BOHR
            r2 = db * db
            r6 = r2 * r2 * r2
            r8 = r6 * r2
            tmp = (A1 * 1.7320508075688772) * rr4_v + A2
            t2 = tmp * tmp
            t6 = t2 * t2 * t2
            t8 = t6 * t2
            ev = (-0.5 * S6) * c6 / (r6 + t6) + (-0.5 * S8) * c8 / (r8 + t8)
            e_b[pl.ds(p * C2 + v * L, L)] = ev
            sii[pl.ds(p * C2 + v * L, L)] = ii_v

        def sc_dma(pp):
            return pltpu.make_async_copy(
                e_b.at[pl.ds(pp * C2, C2)],
                e_sh.at[sii.at[pl.ds(pp * C2, C2)]], ssem)

        @pl.when(g > 0)
        def _():
            sc_dma(1 - p).wait()

        sc_dma(p).start(add=True)
        return x

    lax.fori_loop(0, NCH2, chunk, 0)

    pf = jnp.bitwise_and(NCH2 - 1, 1)
    pltpu.make_async_copy(
        e_b.at[pl.ds(pf * C2, C2)],
        e_sh.at[sii.at[pl.ds(pf * C2, C2)]], ssem).wait()
    plsc.subcore_barrier()
    sl = pl.ds(s * SLICE, SLICE)
    pltpu.sync_copy(e_sh.at[sl], zb)
    pltpu.sync_copy(zb, e_out.at[pl.ds(c * NPAD + s * SLICE, SLICE)])


@functools.partial(
    pl.kernel,
    out_type=jax.ShapeDtypeStruct((NPAD,), jnp.float32),
    mesh=_mesh,
    compiler_params=pltpu.CompilerParams(needs_layout_passes=False),
    scratch_types=dict(
        b0=pltpu.VMEM((CSL,), jnp.float32),
        b1=pltpu.VMEM((CSL,), jnp.float32),
    ),
)
def _phase_c(e_part, e_out, b0, b1):
    w = lax.axis_index("c") * NS + lax.axis_index("s")
    pltpu.sync_copy(e_part.at[pl.ds(w * CSL, CSL)], b0)
    pltpu.sync_copy(e_part.at[pl.ds(NPAD + w * CSL, CSL)], b1)

    def add16(i, x):
        s16 = pl.ds(i * L, L)
        b0[s16] = b0[s16] + b1[s16]
        return x

    lax.fori_loop(0, CSL // L, add16, 0)
    pltpu.sync_copy(b0, e_out.at[pl.ds(w * CSL, CSL)])


def kernel(Za, Dij, idx_i, idx_j, c6ab, rcov, r2r4):
    comp = c6ab.reshape(NPAIR, D3_MAXC * D3_MAXC, 3)
    c6t = comp[:, :, 0]
    cni_t = comp[:, :, 1]
    cnj_t = comp[:, :, 2]
    pad7 = ((0, 0), (0, ROWW - 25))
    t_c6 = jnp.pad(c6t, pad7)
    t_x = jnp.pad((-2.0 * K3) * cni_t, pad7)
    t_y = jnp.pad((-2.0 * K3) * cnj_t, pad7)
    t_b = jnp.pad(K3 * (cni_t * cni_t + cnj_t * cnj_t), pad7)
    rcov_p = jnp.pad(rcov, (0, 1))
    r2r4_p = jnp.pad(jnp.sqrt(r2r4), (0, 1))
    nc_part, pid, rr4 = _phase_a(Za, Dij, idx_i, idx_j, rcov_p, r2r4_p)
    nc_full = _phase_c(nc_part)
    e_part = _phase_b(Dij, idx_i, idx_j, pid, rr4,
                      t_c6, t_x, t_y, t_b, nc_full)
    e_full = _phase_c(e_part)
    return e_full[:N_NODES]


# speedup vs baseline: 54.4089x; 1.0537x over previous
import functools

import jax
import jax.numpy as jnp
from jax import lax
from jax.experimental import pallas as pl
from jax.experimental.pallas import tpu as pltpu
from jax.experimental.pallas import tpu_sc as plsc

D3_MAXC = 5
NZ = 95
NPAIR = NZ * NZ
K1 = 16.0
K3 = -4.0
S6 = 1.0
S8 = 0.9171
A1 = 0.3385
A2 = 2.883
INV_BOHR = 1.0 / 0.5291772108
N_NODES = 100000
N_EDGES = 1600000

NC = 2
NS = 16
NW = NC * NS
L = 16

NPAD = 100352
SLICE = NPAD // NS
CSL = NPAD // NW
EPW = N_EDGES // NW

C1 = 400
NCH1 = EPW // C1
C2 = 80
NCH2 = EPW // C2
FROW = 128

_mesh = plsc.VectorSubcoreMesh(core_axis_name="c", subcore_axis_name="s")


def _cvec(n):
    return jnp.full((L,), n, jnp.int32)


def _zero_fill(buf, nwords):
    z = jnp.zeros((L,), jnp.float32)

    def body(i, x):
        buf[pl.ds(i * L, L)] = z
        return x

    lax.fori_loop(0, nwords // L, body, 0)


@functools.partial(
    pl.kernel,
    out_type=(
        jax.ShapeDtypeStruct((NC * NPAD,), jnp.float32),
        jax.ShapeDtypeStruct((N_EDGES,), jnp.int32),
        jax.ShapeDtypeStruct((N_EDGES,), jnp.float32),
    ),
    mesh=_mesh,
    compiler_params=pltpu.CompilerParams(needs_layout_passes=False),
    scratch_types=dict(
        za_v=pltpu.VMEM((N_NODES,), jnp.int32),
        rcv=pltpu.VMEM((96,), jnp.float32),
        r4v=pltpu.VMEM((96,), jnp.float32),
        ii_b=pltpu.VMEM((2 * C1,), jnp.int32),
        jj_b=pltpu.VMEM((2 * C1,), jnp.int32),
        d_b=pltpu.VMEM((2 * C1,), jnp.float32),
        pid_b=pltpu.VMEM((2 * C1,), jnp.int32),
        rr4_b=pltpu.VMEM((2 * C1,), jnp.float32),
        damp_b=pltpu.VMEM((2 * C1,), jnp.float32),
        sii=pltpu.VMEM((2 * C1,), jnp.int32),
        zb=pltpu.VMEM((SLICE,), jnp.float32),
        nc_sh=pltpu.VMEM_SHARED((NPAD,), jnp.float32),
        lsem=pltpu.SemaphoreType.DMA,
        osem=pltpu.SemaphoreType.DMA,
        ssem=pltpu.SemaphoreType.DMA,
    ),
)
def _phase_a(za, dij, idx_i, idx_j, rcov, r2r4, nc_out, pid_out, rr4_out,
             za_v, rcv, r4v, ii_b, jj_b, d_b, pid_b, rr4_b, damp_b, sii, zb,
             nc_sh, lsem, osem, ssem):
    c = lax.axis_index("c")
    s = lax.axis_index("s")
    base_e = (c * NS + s) * EPW

    _zero_fill(zb, SLICE)
    pltpu.sync_copy(zb, nc_sh.at[pl.ds(s * SLICE, SLICE)])
    pltpu.sync_copy(za, za_v)
    pltpu.sync_copy(rcov, rcv)
    pltpu.sync_copy(r2r4, r4v)
    plsc.subcore_barrier()

    def lin_dma(g, slot):
        src = pl.ds(base_e + g * C1, C1)
        dst = pl.ds(slot * C1, C1)
        return (
            pltpu.make_async_copy(idx_i.at[src], ii_b.at[dst], lsem),
            pltpu.make_async_copy(idx_j.at[src], jj_b.at[dst], lsem),
            pltpu.make_async_copy(dij.at[src], d_b.at[dst], lsem),
        )

    for d in lin_dma(0, 0):
        d.start()

    def chunk(g, x):
        p = jnp.bitwise_and(g, 1)
        for d in lin_dma(g, p):
            d.wait()

        @pl.when(g + 1 < NCH1)
        def _():
            for d in lin_dma(g + 1, 1 - p):
                d.start()

        for g5 in range(C1 // L // 5):
            offs = [p * C1 + (g5 * 5 + t) * L for t in range(5)]
            iis = [ii_b[pl.ds(o, L)] for o in offs]
            jjs = [jj_b[pl.ds(o, L)] for o in offs]
            dvs = [d_b[pl.ds(o, L)] for o in offs]
            zis = [plsc.load_gather(za_v, [x]) for x in iis]
            zjs = [plsc.load_gather(za_v, [x]) for x in jjs]
            rcis = [plsc.load_gather(rcv, [z]) for z in zis]
            rcjs = [plsc.load_gather(rcv, [z]) for z in zjs]
            r4is = [plsc.load_gather(r4v, [z]) for z in zis]
            r4js = [plsc.load_gather(r4v, [z]) for z in zjs]
            for t in range(5):
                o = offs[t]
                rr = (rcis[t] + rcjs[t]) / (dvs[t] * INV_BOHR)
                damp = 1.0 / (1.0 + jnp.exp(K1 - K1 * rr))
                pid_b[pl.ds(o, L)] = zis[t] * NZ + zjs[t]
                rr4_b[pl.ds(o, L)] = r4is[t] * r4js[t]
                damp_b[pl.ds(o, L)] = damp
                sii[pl.ds(o, L)] = iis[t]

        def out_dma(gg, pp):
            dst = pl.ds(base_e + gg * C1, C1)
            return (
                pltpu.make_async_copy(
                    pid_b.at[pl.ds(pp * C1, C1)], pid_out.at[dst], osem),
                pltpu.make_async_copy(
                    rr4_b.at[pl.ds(pp * C1, C1)], rr4_out.at[dst], osem),
            )

        def sc_dma(pp):
            return pltpu.make_async_copy(
                damp_b.at[pl.ds(pp * C1, C1)],
                nc_sh.at[sii.at[pl.ds(pp * C1, C1)]], ssem)

        @pl.when(g > 0)
        def _():
            for d in out_dma(g - 1, 1 - p):
                d.wait()
            sc_dma(1 - p).wait()

        for d in out_dma(g, p):
            d.start()
        sc_dma(p).start(add=True)
        return x

    lax.fori_loop(0, NCH1, chunk, 0)

    pf = jnp.bitwise_and(NCH1 - 1, 1)
    dst = pl.ds(base_e + (NCH1 - 1) * C1, C1)
    pltpu.make_async_copy(
        pid_b.at[pl.ds(pf * C1, C1)], pid_out.at[dst], osem).wait()
    pltpu.make_async_copy(
        rr4_b.at[pl.ds(pf * C1, C1)], rr4_out.at[dst], osem).wait()
    pltpu.make_async_copy(
        damp_b.at[pl.ds(pf * C1, C1)],
        nc_sh.at[sii.at[pl.ds(pf * C1, C1)]], ssem).wait()
    plsc.subcore_barrier()
    sl = pl.ds(s * SLICE, SLICE)
    pltpu.sync_copy(nc_sh.at[sl], zb)
    pltpu.sync_copy(zb, nc_out.at[pl.ds(c * NPAD + s * SLICE, SLICE)])


@functools.partial(
    pl.kernel,
    out_type=jax.ShapeDtypeStruct((NC * NPAD,), jnp.float32),
    mesh=_mesh,
    compiler_params=pltpu.CompilerParams(
        needs_layout_passes=False, use_tc_tiling_on_sc=False),
    scratch_types=dict(
        nc_v=pltpu.VMEM((NPAD,), jnp.float32),
        r_tab=pltpu.VMEM((2 * C2, FROW), jnp.float32),
        ii_b=pltpu.VMEM((3 * C2,), jnp.int32),
        jj_b=pltpu.VMEM((3 * C2,), jnp.int32),
        d_b=pltpu.VMEM((3 * C2,), jnp.float32),
        rr4_b=pltpu.VMEM((3 * C2,), jnp.float32),
        pid_b=pltpu.VMEM((3 * C2,), jnp.int32),
        e_b=pltpu.VMEM((2 * C2,), jnp.float32),
        sii=pltpu.VMEM((2 * C2,), jnp.int32),
        zb=pltpu.VMEM((SLICE // 8,), jnp.float32),
        e_sh=pltpu.VMEM_SHARED((NPAD,), jnp.float32),
        lsem=pltpu.SemaphoreType.DMA,
        gsem=pltpu.SemaphoreType.DMA,
        ssem=pltpu.SemaphoreType.DMA,
    ),
)
def _phase_b(dij, idx_i, idx_j, pid, rr4, tab, nc_full, e_out,
             nc_v, r_tab, ii_b, jj_b, d_b, rr4_b,
             pid_b, e_b, sii, zb, e_sh, lsem, gsem, ssem):
    c = lax.axis_index("c")
    s = lax.axis_index("s")
    base_e = (c * NS + s) * EPW

    _zero_fill(zb, SLICE // 8)
    for u in range(8):
        pltpu.sync_copy(
            zb, e_sh.at[pl.ds(s * SLICE + u * (SLICE // 8), SLICE // 8)])
    pltpu.sync_copy(nc_full, nc_v)
    plsc.subcore_barrier()

    def lin_dma(g, slot):
        src = pl.ds(base_e + g * C2, C2)
        dst = pl.ds(slot * C2, C2)
        return (
            pltpu.make_async_copy(idx_i.at[src], ii_b.at[dst], lsem),
            pltpu.make_async_copy(idx_j.at[src], jj_b.at[dst], lsem),
            pltpu.make_async_copy(dij.at[src], d_b.at[dst], lsem),
            pltpu.make_async_copy(rr4.at[src], rr4_b.at[dst], lsem),
            pltpu.make_async_copy(pid.at[src], pid_b.at[dst], lsem),
        )

    lane = lax.iota(jnp.int32, L)

    def gat_dma(slot, p):
        isl = pid_b.at[pl.ds(slot * C2, C2)]
        dsl = pl.ds(p * C2, C2)
        return (
            pltpu.make_async_copy(tab.at[isl], r_tab.at[dsl], gsem),
        )

    for d in lin_dma(0, 0):
        d.start()
    for d in lin_dma(0, 0):
        d.wait()
    for d in gat_dma(0, 0):
        d.start()
    for d in lin_dma(1, 1):
        d.start()

    def chunk(g, x):
        p = jnp.bitwise_and(g, 1)
        q = lax.rem(g, 3)
        for d in gat_dma(q, p):
            d.wait()

        @pl.when(g + 1 < NCH2)
        def _():
            q1 = lax.rem(g + 1, 3)
            for d in lin_dma(g + 1, q1):
                d.wait()
            for d in gat_dma(q1, 1 - p):
                d.start()

        @pl.when(g + 2 < NCH2)
        def _():
            for d in lin_dma(g + 2, lax.rem(g + 2, 3)):
                d.start()

        for v in range(C2 // L):
            off = q * C2 + v * L
            ii_v = ii_b[pl.ds(off, L)]
            jj_v = jj_b[pl.ds(off, L)]
            d_v = d_b[pl.ds(off, L)]
            rr4_v = rr4_b[pl.ds(off, L)]
            nci = plsc.load_gather(nc_v, [ii_v])
            ncj = plsc.load_gather(nc_v, [jj_v])
            row = lane + (p * C2 + v * L)

            tks = []
            tmaxs = [None] * 4
            for k in range(25):
                xk = plsc.load_gather(r_tab, [row, _cvec(32 + k)])
                yk = plsc.load_gather(r_tab, [row, _cvec(64 + k)])
                bk = plsc.load_gather(r_tab, [row, _cvec(96 + k)])
                tk = (bk + xk * nci) + yk * ncj
                tks.append(tk)
                a = k & 3
                tmaxs[a] = tk if tmaxs[a] is None else jnp.maximum(tmaxs[a], tk)
            sh = jnp.maximum(jnp.maximum(tmaxs[0], tmaxs[1]),
                             jnp.maximum(tmaxs[2], tmaxs[3]))
            ses = [None] * 4
            scs = [None] * 4
            for k in range(25):
                e = jnp.exp(tks[k] - sh)
                c6k = plsc.load_gather(r_tab, [row, _cvec(k)])
                ec = e * c6k
                a = k & 3
                ses[a] = e if ses[a] is None else ses[a] + e
                scs[a] = ec if scs[a] is None else scs[a] + ec
            se = (ses[0] + ses[1]) + (ses[2] + ses[3])
            sc6 = (scs[0] + scs[1]) + (scs[2] + scs[3])
            c6 = sc6 / se
            c8 = 3.0 * c6 * (rr4_v * rr4_v)
            db = d_v * INV_BOHR
            r2 = db * db
            r6 = r2 * r2 * r2
            r8 = r6 * r2
            tmp = (A1 * 1.7320508075688772) * rr4_v + A2
            t2 = tmp * tmp
            t6 = t2 * t2 * t2
            t8 = t6 * t2
            ev = (-0.5 * S6) * c6 / (r6 + t6) + (-0.5 * S8) * c8 / (r8 + t8)
            e_b[pl.ds(p * C2 + v * L, L)] = ev
            sii[pl.ds(p * C2 + v * L, L)] = ii_v

        def sc_dma(pp):
            return pltpu.make_async_copy(
                e_b.at[pl.ds(pp * C2, C2)],
                e_sh.at[sii.at[pl.ds(pp * C2, C2)]], ssem)

        @pl.when(g > 0)
        def _():
            sc_dma(1 - p).wait()

        sc_dma(p).start(add=True)
        return x

    lax.fori_loop(0, NCH2, chunk, 0)

    pf = jnp.bitwise_and(NCH2 - 1, 1)
    pltpu.make_async_copy(
        e_b.at[pl.ds(pf * C2, C2)],
        e_sh.at[sii.at[pl.ds(pf * C2, C2)]], ssem).wait()
    plsc.subcore_barrier()
    pltpu.sync_copy(e_sh.at[pl.ds(s * SLICE, SLICE)],
                    e_out.at[pl.ds(c * NPAD + s * SLICE, SLICE)])


@functools.partial(
    pl.kernel,
    out_type=jax.ShapeDtypeStruct((NPAD,), jnp.float32),
    mesh=_mesh,
    compiler_params=pltpu.CompilerParams(needs_layout_passes=False),
    scratch_types=dict(
        b0=pltpu.VMEM((CSL,), jnp.float32),
        b1=pltpu.VMEM((CSL,), jnp.float32),
    ),
)
def _phase_c(e_part, e_out, b0, b1):
    w = lax.axis_index("c") * NS + lax.axis_index("s")
    pltpu.sync_copy(e_part.at[pl.ds(w * CSL, CSL)], b0)
    pltpu.sync_copy(e_part.at[pl.ds(NPAD + w * CSL, CSL)], b1)

    def add16(i, x):
        s16 = pl.ds(i * L, L)
        b0[s16] = b0[s16] + b1[s16]
        return x

    lax.fori_loop(0, CSL // L, add16, 0)
    pltpu.sync_copy(b0, e_out.at[pl.ds(w * CSL, CSL)])


def kernel(Za, Dij, idx_i, idx_j, c6ab, rcov, r2r4):
    comp = c6ab.reshape(NPAIR, D3_MAXC * D3_MAXC, 3)
    c6t = comp[:, :, 0]
    cni_t = comp[:, :, 1]
    cnj_t = comp[:, :, 2]
    pad7 = ((0, 0), (0, 7))
    tab = jnp.concatenate(
        [jnp.pad(c6t, pad7),
         jnp.pad((-2.0 * K3) * cni_t, pad7),
         jnp.pad((-2.0 * K3) * cnj_t, pad7),
         jnp.pad(K3 * (cni_t * cni_t + cnj_t * cnj_t), pad7)], axis=1)
    rcov_p = jnp.pad(rcov, (0, 1))
    r2r4_p = jnp.pad(jnp.sqrt(r2r4), (0, 1))
    nc_part, pid, rr4 = _phase_a(Za, Dij, idx_i, idx_j, rcov_p, r2r4_p)
    nc_full = _phase_c(nc_part)
    e_part = _phase_b(Dij, idx_i, idx_j, pid, rr4, tab, nc_full)
    e_full = _phase_c(e_part)
    return e_full[:N_NODES]
